# Initial kernel scaffold; baseline (speedup 1.0000x reference)
import jax
import jax.numpy as jnp
from jax.experimental import pallas as pl


def kernel(hidden_states, weight, e_score_correction_bias):
    raise NotImplementedError("write your pallas kernel here")


# speedup vs baseline: 5.1376x; 5.1376x over previous
import functools

import jax
import jax.numpy as jnp
import numpy as np
from jax.experimental import pallas as pl
from jax.experimental.pallas import tpu as pltpu

_E = 64
_NG = 16
_PG = 4
_TOPK = 8
_TOPKG = 4
_SCALE = 2.5
_NEG = float("-inf")


def _gate_body(x_ref, w_ref, b_ref, idx_ref, wgt_ref):
    logits = jax.lax.dot_general(
        w_ref[...], x_ref[...],
        (((1,), (1,)), ((), ())),
        preferred_element_type=jnp.float32,
    )
    s = 1.0 / (1.0 + jnp.exp(-logits))
    sfc = s + b_ref[...]

    tile = s.shape[1]
    b0 = sfc[0:16, :]
    b1 = sfc[16:32, :]
    b2 = sfc[32:48, :]
    b3 = sfc[48:64, :]
    gs = jnp.maximum(b0 + b1, b0 + b2)
    gs = jnp.maximum(gs, b0 + b3)
    gs = jnp.maximum(gs, b1 + b2)
    gs = jnp.maximum(gs, b1 + b3)
    gs = jnp.maximum(gs, b2 + b3)

    giota = jax.lax.broadcasted_iota(jnp.int32, (16, tile), 0)
    gmask = jnp.zeros((16, tile), dtype=jnp.bool_)
    work = gs
    for _ in range(_TOPKG):
        m = jnp.max(work, axis=0, keepdims=True)
        cand = jnp.where(work == m, giota, _NG)
        gsel = jnp.min(cand, axis=0, keepdims=True)
        hit = giota == gsel
        gmask = gmask | hit
        work = jnp.where(hit, _NEG, work)

    emask = jnp.concatenate([gmask, gmask, gmask, gmask], axis=0)
    masked = jnp.where(emask, sfc, _NEG)

    r = jax.lax.broadcasted_iota(jnp.int32, (_E, tile), 0)
    orig = 4 * (r & 15) + (r >> 4)

    idx_rows = []
    wgt_rows = []
    for _ in range(_TOPK):
        m = jnp.max(masked, axis=0, keepdims=True)
        ci = jnp.where(masked == m, orig, _E)
        sel_id = jnp.min(ci, axis=0, keepdims=True)
        sel = orig == sel_id
        w_k = jnp.sum(jnp.where(sel, s, 0.0), axis=0, keepdims=True)
        masked = jnp.where(sel, _NEG, masked)
        idx_rows.append(sel_id)
        wgt_rows.append(w_k)

    wgt = jnp.concatenate(wgt_rows, axis=0)
    denom = jnp.sum(wgt, axis=0, keepdims=True) + 1e-20
    idx_ref[...] = jnp.concatenate(idx_rows, axis=0)
    wgt_ref[...] = wgt * (_SCALE / denom)


@functools.partial(jax.jit, static_argnums=())
def _gate(x, w_perm, b_perm):
    t, h = x.shape
    tile = 512
    grid = t // tile
    idx_t, wgt_t = pl.pallas_call(
        _gate_body,
        grid=(grid,),
        in_specs=[
            pl.BlockSpec((tile, h), lambda i: (i, 0)),
            pl.BlockSpec((_E, h), lambda i: (0, 0)),
            pl.BlockSpec((_E, 1), lambda i: (0, 0)),
        ],
        out_specs=[
            pl.BlockSpec((_TOPK, tile), lambda i: (0, i)),
            pl.BlockSpec((_TOPK, tile), lambda i: (0, i)),
        ],
        out_shape=[
            jax.ShapeDtypeStruct((_TOPK, t), jnp.int32),
            jax.ShapeDtypeStruct((_TOPK, t), jnp.float32),
        ],
        compiler_params=pltpu.CompilerParams(
            dimension_semantics=("arbitrary",),
        ),
    )(x, w_perm, b_perm)
    return idx_t.T, wgt_t.T


_PERM = np.array([4 * g + p for p in range(_PG) for g in range(_NG)], dtype=np.int32)


def kernel(hidden_states, weight, e_score_correction_bias):
    bsz, seq_len, h = hidden_states.shape
    x = hidden_states.reshape(bsz * seq_len, h).astype(jnp.float32)
    w_perm = weight.astype(jnp.float32)[_PERM]
    b_perm = e_score_correction_bias.astype(jnp.float32)[_PERM][:, None]
    return _gate(x, w_perm, b_perm)


# speedup vs baseline: 5.4896x; 1.0685x over previous
import functools

import jax
import jax.numpy as jnp
import numpy as np
from jax.experimental import pallas as pl
from jax.experimental.pallas import tpu as pltpu

_E = 64
_NG = 16
_PG = 4
_TOPK = 8
_TOPKG = 4
_SCALE = 2.5
_NEG = float("-inf")


def _gate_body(x_ref, w_ref, b_ref, idx_ref, wgt_ref):
    logits = jax.lax.dot_general(
        w_ref[...], x_ref[...],
        (((1,), (1,)), ((), ())),
        preferred_element_type=jnp.float32,
    )
    s = 1.0 / (1.0 + jnp.exp(-logits))
    sfc = s + b_ref[...]

    tile = s.shape[1]
    b0 = sfc[0:16, :]
    b1 = sfc[16:32, :]
    b2 = sfc[32:48, :]
    b3 = sfc[48:64, :]
    gs = jnp.maximum(b0 + b1, b0 + b2)
    gs = jnp.maximum(gs, b0 + b3)
    gs = jnp.maximum(gs, b1 + b2)
    gs = jnp.maximum(gs, b1 + b3)
    gs = jnp.maximum(gs, b2 + b3)

    giota = jax.lax.broadcasted_iota(jnp.int32, (16, tile), 0)
    gmask = jnp.zeros((16, tile), dtype=jnp.bool_)
    work = gs
    for _ in range(_TOPKG):
        m = jnp.max(work, axis=0, keepdims=True)
        cand = jnp.where(work == m, giota, _NG)
        gsel = jnp.min(cand, axis=0, keepdims=True)
        hit = giota == gsel
        gmask = gmask | hit
        work = jnp.where(hit, _NEG, work)

    emask = jnp.concatenate([gmask, gmask, gmask, gmask], axis=0)
    masked = jnp.where(emask, sfc, _NEG)

    r = jax.lax.broadcasted_iota(jnp.int32, (_E, tile), 0)
    orig = 4 * (r & 15) + (r >> 4)

    idx_rows = []
    wgt_rows = []
    for _ in range(_TOPK):
        m = jnp.max(masked, axis=0, keepdims=True)
        ci = jnp.where(masked == m, orig, _E)
        sel_id = jnp.min(ci, axis=0, keepdims=True)
        sel = orig == sel_id
        w_k = jnp.sum(jnp.where(sel, s, 0.0), axis=0, keepdims=True)
        masked = jnp.where(sel, _NEG, masked)
        idx_rows.append(sel_id)
        wgt_rows.append(w_k)

    wgt = jnp.concatenate(wgt_rows, axis=0)
    denom = jnp.sum(wgt, axis=0, keepdims=True) + 1e-20
    idx_ref[...] = jnp.concatenate(idx_rows, axis=0)
    wgt_ref[...] = wgt * (_SCALE / denom)


@functools.partial(jax.jit, static_argnums=())
def _gate(x, w_perm, b_perm):
    t, h = x.shape
    tile = 1024
    grid = t // tile
    idx_t, wgt_t = pl.pallas_call(
        _gate_body,
        grid=(grid,),
        in_specs=[
            pl.BlockSpec((tile, h), lambda i: (i, 0)),
            pl.BlockSpec((_E, h), lambda i: (0, 0)),
            pl.BlockSpec((_E, 1), lambda i: (0, 0)),
        ],
        out_specs=[
            pl.BlockSpec((_TOPK, tile), lambda i: (0, i)),
            pl.BlockSpec((_TOPK, tile), lambda i: (0, i)),
        ],
        out_shape=[
            jax.ShapeDtypeStruct((_TOPK, t), jnp.int32),
            jax.ShapeDtypeStruct((_TOPK, t), jnp.float32),
        ],
        compiler_params=pltpu.CompilerParams(
            dimension_semantics=("arbitrary",),
        ),
    )(x, w_perm, b_perm)
    return idx_t.T, wgt_t.T


_PERM = np.array([4 * g + p for p in range(_PG) for g in range(_NG)], dtype=np.int32)


def kernel(hidden_states, weight, e_score_correction_bias):
    bsz, seq_len, h = hidden_states.shape
    x = hidden_states.reshape(bsz * seq_len, h).astype(jnp.float32)
    w_perm = weight.astype(jnp.float32)[_PERM]
    b_perm = e_score_correction_bias.astype(jnp.float32)[_PERM][:, None]
    return _gate(x, w_perm, b_perm)
